# baseline (speedup 1.0000x reference)
import functools

import jax
import jax.numpy as jnp
from jax.experimental import pallas as pl
from jax.experimental.pallas import tpu as pltpu

E = 8
D = 768
T = 2048
BT = 256


def _tables_body(ins_ref, wbeta_ref, wgam_ref, wr_ref, be_ref,
                 table2_ref, gam_ref, rgam_ref):
    ins = ins_ref[0]
    m = jnp.mean(ins, axis=0, keepdims=True)
    beta = jnp.sum(wbeta_ref[0] * m, axis=1)
    gamma = jnp.sum(wgam_ref[0, 0] * m[0])
    rgam = jnp.sum(wr_ref[0, 0] * m[0])
    table2_ref[0, 0, :] = gamma * be_ref[0, 0] + beta
    gam_ref[...] = jnp.full((1, 1, 128), gamma, dtype=jnp.float32)
    rgam_ref[...] = jnp.full((1, 1, 128), rgam, dtype=jnp.float32)


def _compute_tables(Ins_tk, Wbeta, Wgam, Wr, be):
    return pl.pallas_call(
        _tables_body,
        grid=(E,),
        in_specs=[
            pl.BlockSpec((1, 32, D), lambda e: (0, 0, 0)),
            pl.BlockSpec((1, D, D), lambda e: (e, 0, 0)),
            pl.BlockSpec((1, 1, D), lambda e: (e, 0, 0)),
            pl.BlockSpec((1, 1, D), lambda e: (e, 0, 0)),
            pl.BlockSpec((1, 1, D), lambda e: (e, 0, 0)),
        ],
        out_specs=[
            pl.BlockSpec((1, 1, D), lambda e: (e, 0, 0)),
            pl.BlockSpec((1, 1, 128), lambda e: (e, 0, 0)),
            pl.BlockSpec((1, 1, 128), lambda e: (e, 0, 0)),
        ],
        out_shape=[
            jax.ShapeDtypeStruct((E, 1, D), jnp.float32),
            jax.ShapeDtypeStruct((E, 1, 128), jnp.float32),
            jax.ShapeDtypeStruct((E, 1, 128), jnp.float32),
        ],
    )(Ins_tk, Wbeta, Wgam.reshape(E, 1, D), Wr.reshape(E, 1, D),
      be.reshape(E, 1, D))


def _moe_body(x_ref, wg_ref, gam_ref, rgam_ref, table2_ref, we_ref,
              out_ref, comb_s, aa_s):
    e = pl.program_id(1)

    @pl.when(e == 0)
    def _router():
        xb = x_ref[...]
        gam_row = gam_ref[:, 0, 0:1].reshape(1, E)
        rgam_row = rgam_ref[:, 0, 0:1].reshape(1, E)
        logits = jax.lax.dot_general(
            xb, wg_ref[...], (((1,), (1,)), ((), ())),
            preferred_element_type=jnp.float32) + rgam_row
        mx = jnp.max(logits, axis=1, keepdims=True)
        w = 1.0 / jnp.sum(jnp.exp(logits - mx), axis=1, keepdims=True)
        eidx = jnp.argmax(logits, axis=1)
        onehot = (jax.lax.broadcasted_iota(jnp.int32, (BT, E), 1)
                  == eidx[:, None]).astype(jnp.float32)
        comb_s[...] = w * onehot
        aa_s[...] = comb_s[...] * gam_row

    xb = x_ref[...]
    y = jax.lax.dot_general(
        xb, we_ref[0], (((1,), (1,)), ((), ())),
        preferred_element_type=jnp.float32)
    sel = (jax.lax.broadcasted_iota(jnp.int32, (BT, E), 1) == e)
    aa_col = jnp.sum(jnp.where(sel, aa_s[...], 0.0), axis=1, keepdims=True)
    comb_col = jnp.sum(jnp.where(sel, comb_s[...], 0.0), axis=1, keepdims=True)
    contrib = aa_col * y + comb_col * table2_ref[0, 0][None, :]

    @pl.when(e == 0)
    def _init():
        out_ref[...] = contrib

    @pl.when(e != 0)
    def _acc():
        out_ref[...] += contrib


def kernel(x, Ins_tk, Wg, We, be, Wgam, Wbeta, Wr):
    B, C, L = x.shape
    xf = x.reshape(T, D)
    table2, gam, rgam = _compute_tables(Ins_tk, Wbeta, Wgam, Wr, be)
    out = pl.pallas_call(
        _moe_body,
        grid=(T // BT, E),
        in_specs=[
            pl.BlockSpec((BT, D), lambda t, e: (t, 0)),
            pl.BlockSpec((E, D), lambda t, e: (0, 0)),
            pl.BlockSpec((E, 1, 128), lambda t, e: (0, 0, 0)),
            pl.BlockSpec((E, 1, 128), lambda t, e: (0, 0, 0)),
            pl.BlockSpec((1, 1, D), lambda t, e: (e, 0, 0)),
            pl.BlockSpec((1, D, D), lambda t, e: (e, 0, 0)),
        ],
        out_specs=pl.BlockSpec((BT, D), lambda t, e: (t, 0)),
        out_shape=jax.ShapeDtypeStruct((T, D), jnp.float32),
        scratch_shapes=[
            pltpu.VMEM((BT, E), jnp.float32),
            pltpu.VMEM((BT, E), jnp.float32),
        ],
    )(xf, Wg, gam, rgam, table2, We)
    return out.reshape(B, C, D)


# speedup vs baseline: 1.1018x; 1.1018x over previous
import functools

import jax
import jax.numpy as jnp
from jax import lax
from jax.experimental import pallas as pl
from jax.experimental.pallas import tpu as pltpu
from jax.experimental.pallas import tpu_sc as plsc

E = 8
D = 768
T = 2048
BT = 256
BM = 256
P = T // BM
NLOG = P + E - 1

NC, NS = 2, 16
NW = NC * NS
TW = T // NW


def _tables_body(ins_ref, wbeta_ref, wgam_ref, be_ref, table2_ref, gam_ref):
    ins = ins_ref[0]
    m = jnp.mean(ins, axis=0, keepdims=True)
    beta = jnp.sum(wbeta_ref[0] * m, axis=1)
    gamma = jnp.sum(wgam_ref[0, 0] * m[0])
    table2_ref[0, 0, :] = gamma * be_ref[0, 0] + beta
    gam_ref[...] = jnp.full((1, 1, 128), gamma, dtype=jnp.float32)


def _compute_tables(Ins_tk, Wbeta, Wgam, be):
    return pl.pallas_call(
        _tables_body,
        grid=(E,),
        in_specs=[
            pl.BlockSpec((1, 32, D), lambda e: (0, 0, 0)),
            pl.BlockSpec((1, D, D), lambda e: (e, 0, 0)),
            pl.BlockSpec((1, 1, D), lambda e: (e, 0, 0)),
            pl.BlockSpec((1, 1, D), lambda e: (e, 0, 0)),
        ],
        out_specs=[
            pl.BlockSpec((1, 1, D), lambda e: (e, 0, 0)),
            pl.BlockSpec((1, 1, 128), lambda e: (e, 0, 0)),
        ],
        out_shape=[
            jax.ShapeDtypeStruct((E, 1, D), jnp.float32),
            jax.ShapeDtypeStruct((E, 1, 128), jnp.float32),
        ],
    )(Ins_tk, Wbeta, Wgam.reshape(E, 1, D), be.reshape(E, 1, D))


def _router_body(x_ref, logits_ref,
                 xs_ref, wrep_ref, eidx_ref, rank_ref, hist_ref, hist_s):
    t = pl.program_id(0)

    @pl.when(t == 0)
    def _init():
        hist_s[...] = jnp.zeros_like(hist_s)

    logits = logits_ref[...]
    mx = jnp.max(logits, axis=1, keepdims=True)
    w = 1.0 / jnp.sum(jnp.exp(logits - mx), axis=1, keepdims=True)
    eidx = jnp.argmax(logits, axis=1)
    onehot = (lax.broadcasted_iota(jnp.int32, (BT, E), 1)
              == eidx[:, None]).astype(jnp.float32)

    xs_ref[...] = w * x_ref[...]
    wrep_ref[...] = jnp.broadcast_to(w, (BT, 128))

    ii = lax.broadcasted_iota(jnp.int32, (BT, BT), 0)
    jj = lax.broadcasted_iota(jnp.int32, (BT, BT), 1)
    tri = (ii > jj).astype(jnp.float32)
    rank_local = lax.dot_general(
        tri, onehot, (((1,), (0,)), ((), ())),
        preferred_element_type=jnp.float32)
    rank = jnp.sum((rank_local + hist_s[0:1, :E]) * onehot, axis=1)
    hist_s[0, :E] += jnp.sum(onehot, axis=0)

    eidx_ref[0, 0, :] = eidx
    rank_ref[0, 0, :] = rank.astype(jnp.int32)
    hist_ref[...] = hist_s[...]


def _router(xf, logits):
    return pl.pallas_call(
        _router_body,
        grid=(T // BT,),
        in_specs=[
            pl.BlockSpec((BT, D), lambda t: (t, 0)),
            pl.BlockSpec((BT, E), lambda t: (t, 0)),
        ],
        out_specs=[
            pl.BlockSpec((BT, D), lambda t: (t, 0)),
            pl.BlockSpec((BT, 128), lambda t: (t, 0)),
            pl.BlockSpec((1, 1, BT), lambda t: (t, 0, 0)),
            pl.BlockSpec((1, 1, BT), lambda t: (t, 0, 0)),
            pl.BlockSpec((1, 128), lambda t: (0, 0)),
        ],
        out_shape=[
            jax.ShapeDtypeStruct((T, D), jnp.float32),
            jax.ShapeDtypeStruct((T, 128), jnp.float32),
            jax.ShapeDtypeStruct((T // BT, 1, BT), jnp.int32),
            jax.ShapeDtypeStruct((T // BT, 1, BT), jnp.int32),
            jax.ShapeDtypeStruct((1, 128), jnp.float32),
        ],
        scratch_shapes=[pltpu.VMEM((1, 128), jnp.float32)],
    )(xf, logits)


def _pos_body(eidx_ref, rank_ref, offp_ref, pos_ref):
    er = eidx_ref[:, 0, :]
    acc = jnp.zeros(er.shape, jnp.float32)
    for e in range(E):
        acc = acc + jnp.where(er == e, offp_ref[0, e], 0.0)
    pos_ref[:, 0, :] = acc.astype(jnp.int32) + rank_ref[:, 0, :]


def _compute_pos(eidx3, rank3, offp):
    return pl.pallas_call(
        _pos_body,
        out_shape=jax.ShapeDtypeStruct((T // BT, 1, BT), jnp.int32),
    )(eidx3, rank3, offp)


def _scatter_body(xs_hbm, wrep_hbm, pos_hbm, xsort_hbm, wsort_hbm,
                  pos_v, rows_v, wrow_v, sem):
    wid = lax.axis_index("s") * NC + lax.axis_index("c")
    base = wid * TW
    pltpu.sync_copy(pos_hbm.at[pl.ds(base, TW)], pos_v)
    pltpu.sync_copy(xs_hbm.at[pl.ds(base, TW)], rows_v)
    pltpu.sync_copy(wrep_hbm.at[pl.ds(base, TW)], wrow_v)
    cp1 = pltpu.async_copy(rows_v, xsort_hbm.at[pos_v], sem)
    cp2 = pltpu.async_copy(wrow_v, wsort_hbm.at[pos_v], sem)
    cp1.wait()
    cp2.wait()


@functools.lru_cache(maxsize=None)
def _get_dispatch():
    mesh = plsc.VectorSubcoreMesh(core_axis_name="c", subcore_axis_name="s")
    return pl.kernel(
        _scatter_body,
        mesh=mesh,
        out_type=[
            jax.ShapeDtypeStruct((T, D), jnp.float32),
            jax.ShapeDtypeStruct((T, 128), jnp.float32),
        ],
        scratch_types=[
            pltpu.VMEM((TW,), jnp.int32),
            pltpu.VMEM((TW, D), jnp.float32),
            pltpu.VMEM((TW, 128), jnp.float32),
            pltpu.SemaphoreType.DMA,
        ],
    )


def _gmm_body(tg_ref, tp_ref, rlo_ref, rhi_ref,
              xs_ref, wsort_ref, we_ref, gam_ref, t2_ref, out_ref):
    i = pl.program_id(0)
    phys = tp_ref[i]
    lo = rlo_ref[i]
    hi = rhi_ref[i]
    rows = phys * BM + lax.broadcasted_iota(jnp.int32, (BM, 1), 0)
    mask = (rows >= lo) & (rows < hi)
    xm = jnp.where(mask, xs_ref[...], 0.0).astype(jnp.bfloat16)
    y = lax.dot_general(xm, we_ref[0], (((1,), (1,)), ((), ())),
                        preferred_element_type=jnp.float32)
    gamma = gam_ref[0, 0, 0]
    w_col = wsort_ref[:, 0:1]
    bias = jnp.where(mask, w_col * t2_ref[0, 0][None, :], 0.0)
    contrib = gamma * y + bias
    first = jnp.logical_or(i == 0, phys != tp_ref[jnp.maximum(i - 1, 0)])

    @pl.when(first)
    def _set():
        out_ref[...] = contrib

    @pl.when(jnp.logical_not(first))
    def _acc():
        out_ref[...] += contrib


def _grouped_matmul(tile_g, tile_p, row_lo, row_hi, xs_sorted, wsort,
                    We_bf16, gam, table2_3d):
    grid_spec = pltpu.PrefetchScalarGridSpec(
        num_scalar_prefetch=4,
        grid=(NLOG,),
        in_specs=[
            pl.BlockSpec((BM, D), lambda i, tg, tp, rlo, rhi: (tp[i], 0)),
            pl.BlockSpec((BM, 128), lambda i, tg, tp, rlo, rhi: (tp[i], 0)),
            pl.BlockSpec((1, D, D), lambda i, tg, tp, rlo, rhi: (tg[i], 0, 0)),
            pl.BlockSpec((1, 1, 128), lambda i, tg, tp, rlo, rhi: (tg[i], 0, 0)),
            pl.BlockSpec((1, 1, D), lambda i, tg, tp, rlo, rhi: (tg[i], 0, 0)),
        ],
        out_specs=pl.BlockSpec((BM, D), lambda i, tg, tp, rlo, rhi: (tp[i], 0)),
    )
    return pl.pallas_call(
        _gmm_body,
        grid_spec=grid_spec,
        out_shape=jax.ShapeDtypeStruct((T, D), jnp.float32),
    )(tile_g, tile_p, row_lo, row_hi, xs_sorted, wsort, We_bf16, gam,
      table2_3d)


def _gather_body(ysort_hbm, pos_hbm, out_hbm, pos_v, rows_v, sem):
    wid = lax.axis_index("s") * NC + lax.axis_index("c")
    base = wid * TW
    pltpu.sync_copy(pos_hbm.at[pl.ds(base, TW)], pos_v)
    pltpu.async_copy(ysort_hbm.at[pos_v], rows_v, sem).wait()
    pltpu.sync_copy(rows_v, out_hbm.at[pl.ds(base, TW)])


@functools.lru_cache(maxsize=None)
def _get_undispatch():
    mesh = plsc.VectorSubcoreMesh(core_axis_name="c", subcore_axis_name="s")
    return pl.kernel(
        _gather_body,
        mesh=mesh,
        out_type=jax.ShapeDtypeStruct((T, D), jnp.float32),
        scratch_types=[
            pltpu.VMEM((TW,), jnp.int32),
            pltpu.VMEM((TW, D), jnp.float32),
            pltpu.SemaphoreType.DMA,
        ],
    )


def kernel(x, Ins_tk, Wg, We, be, Wgam, Wbeta, Wr):
    B, C, L = x.shape
    xf = x.reshape(T, D)

    table2_3d, gam = _compute_tables(Ins_tk, Wbeta, Wgam, be)
    router_logits = xf @ Wg.T
    router_gamma = jnp.mean(Ins_tk @ Wr.T, axis=1)[0]
    logits = router_gamma + router_logits

    xs0, wrep, eidx3, rank3, hist = _router(xf, logits)

    sz = hist[0, :E].astype(jnp.int32)
    off = jnp.concatenate([jnp.zeros((1,), jnp.int32),
                           jnp.cumsum(sz)[:-1].astype(jnp.int32)])
    end = off + sz
    t_lo = off // BM
    t_hi = jnp.where(sz > 0, (end - 1) // BM, t_lo - 1)
    n = t_hi - t_lo + 1
    starts = jnp.concatenate([jnp.zeros((1,), jnp.int32),
                              jnp.cumsum(n)[:-1].astype(jnp.int32)])
    total = jnp.sum(n)
    i = jnp.arange(NLOG, dtype=jnp.int32)
    g_i = (jnp.searchsorted(starts, i, side="right") - 1).astype(jnp.int32)
    g_i = jnp.clip(g_i, 0, E - 1)
    phys_i = t_lo[g_i] + (i - starts[g_i])
    valid = i < total
    tile_p = jnp.where(valid, phys_i, P - 1).astype(jnp.int32)
    tile_g = jnp.where(valid, g_i, 0).astype(jnp.int32)
    row_lo = jnp.where(valid, jnp.maximum(off[g_i], phys_i * BM), 0)
    row_hi = jnp.where(valid, jnp.minimum(end[g_i], (phys_i + 1) * BM), 0)

    offp = jnp.zeros((1, 128), jnp.float32).at[0, :E].set(
        off.astype(jnp.float32))
    pos3 = _compute_pos(eidx3, rank3, offp)
    pos = pos3.reshape(T)

    xs_sorted, wsort = _get_dispatch()(xs0, wrep, pos)
    ysorted = _grouped_matmul(tile_g, tile_p,
                              row_lo.astype(jnp.int32),
                              row_hi.astype(jnp.int32),
                              xs_sorted, wsort, We.astype(jnp.bfloat16),
                              gam, table2_3d)
    out = _get_undispatch()(ysorted, pos)
    return out.reshape(B, C, D)


# speedup vs baseline: 1.2592x; 1.1429x over previous
import functools

import jax
import jax.numpy as jnp
from jax import lax
from jax.experimental import pallas as pl
from jax.experimental.pallas import tpu as pltpu
from jax.experimental.pallas import tpu_sc as plsc

E = 8
D = 768
T = 2048
BT = 256
BM = 256
P = T // BM
NLOG = P + E - 1

NC, NS = 2, 16
NW = NC * NS
TW = T // NW


def _tables_body(ins_ref, wbeta_ref, wgam_ref, be_ref, table2_ref, gam_ref):
    ins = ins_ref[0]
    m = jnp.mean(ins, axis=0, keepdims=True)
    beta = jnp.sum(wbeta_ref[0] * m, axis=1)
    gamma = jnp.sum(wgam_ref[0, 0] * m[0])
    table2_ref[0, 0, :] = gamma * be_ref[0, 0] + beta
    gam_ref[...] = jnp.full((1, 1, 128), gamma, dtype=jnp.float32)


def _compute_tables(Ins_tk, Wbeta, Wgam, be):
    return pl.pallas_call(
        _tables_body,
        grid=(E,),
        in_specs=[
            pl.BlockSpec((1, 32, D), lambda e: (0, 0, 0)),
            pl.BlockSpec((1, D, D), lambda e: (e, 0, 0)),
            pl.BlockSpec((1, 1, D), lambda e: (e, 0, 0)),
            pl.BlockSpec((1, 1, D), lambda e: (e, 0, 0)),
        ],
        out_specs=[
            pl.BlockSpec((1, 1, D), lambda e: (e, 0, 0)),
            pl.BlockSpec((1, 1, 128), lambda e: (e, 0, 0)),
        ],
        out_shape=[
            jax.ShapeDtypeStruct((E, 1, D), jnp.float32),
            jax.ShapeDtypeStruct((E, 1, 128), jnp.float32),
        ],
    )(Ins_tk, Wbeta, Wgam.reshape(E, 1, D), be.reshape(E, 1, D))


def _router_body(logits_ref,
                 wrep_ref, eidx_ref, rank_ref, hist_ref, hist_s):
    t = pl.program_id(0)

    @pl.when(t == 0)
    def _init():
        hist_s[...] = jnp.zeros_like(hist_s)

    logits = logits_ref[...]
    mx = jnp.max(logits, axis=1, keepdims=True)
    w = 1.0 / jnp.sum(jnp.exp(logits - mx), axis=1, keepdims=True)
    eidx = jnp.argmax(logits, axis=1)
    onehot = (lax.broadcasted_iota(jnp.int32, (BT, E), 1)
              == eidx[:, None]).astype(jnp.float32)

    wrep_ref[...] = jnp.broadcast_to(w, (BT, 128))

    ii = lax.broadcasted_iota(jnp.int32, (BT, BT), 0)
    jj = lax.broadcasted_iota(jnp.int32, (BT, BT), 1)
    tri = (ii > jj).astype(jnp.float32)
    rank_local = lax.dot_general(
        tri, onehot, (((1,), (0,)), ((), ())),
        preferred_element_type=jnp.float32)
    rank = jnp.sum((rank_local + hist_s[0:1, :E]) * onehot, axis=1)
    hist_s[0, :E] += jnp.sum(onehot, axis=0)

    eidx_ref[0, 0, :] = eidx
    rank_ref[0, 0, :] = rank.astype(jnp.int32)
    hist_ref[...] = hist_s[...]


def _router(logits):
    return pl.pallas_call(
        _router_body,
        grid=(T // BT,),
        in_specs=[
            pl.BlockSpec((BT, E), lambda t: (t, 0)),
        ],
        out_specs=[
            pl.BlockSpec((BT, 128), lambda t: (t, 0)),
            pl.BlockSpec((1, 1, BT), lambda t: (t, 0, 0)),
            pl.BlockSpec((1, 1, BT), lambda t: (t, 0, 0)),
            pl.BlockSpec((1, 128), lambda t: (0, 0)),
        ],
        out_shape=[
            jax.ShapeDtypeStruct((T, 128), jnp.float32),
            jax.ShapeDtypeStruct((T // BT, 1, BT), jnp.int32),
            jax.ShapeDtypeStruct((T // BT, 1, BT), jnp.int32),
            jax.ShapeDtypeStruct((1, 128), jnp.float32),
        ],
        scratch_shapes=[pltpu.VMEM((1, 128), jnp.float32)],
    )(logits)


def _pos_body(eidx_ref, rank_ref, offp_ref, pos_ref):
    er = eidx_ref[:, 0, :]
    acc = jnp.zeros(er.shape, jnp.float32)
    for e in range(E):
        acc = acc + jnp.where(er == e, offp_ref[0, e], 0.0)
    pos_ref[:, 0, :] = acc.astype(jnp.int32) + rank_ref[:, 0, :]


def _compute_pos(eidx3, rank3, offp):
    return pl.pallas_call(
        _pos_body,
        out_shape=jax.ShapeDtypeStruct((T // BT, 1, BT), jnp.int32),
    )(eidx3, rank3, offp)


def _scatter_body(xs_hbm, wrep_hbm, pos_hbm, xsort_hbm, wsort_hbm,
                  pos_v, rows_v, wrow_v, sem):
    wid = lax.axis_index("s") * NC + lax.axis_index("c")
    base = wid * TW
    pltpu.sync_copy(pos_hbm.at[pl.ds(base, TW)], pos_v)
    pltpu.sync_copy(xs_hbm.at[pl.ds(base, TW)], rows_v)
    pltpu.sync_copy(wrep_hbm.at[pl.ds(base, TW)], wrow_v)
    cp1 = pltpu.async_copy(rows_v, xsort_hbm.at[pos_v], sem)
    cp2 = pltpu.async_copy(wrow_v, wsort_hbm.at[pos_v], sem)
    cp1.wait()
    cp2.wait()


@functools.lru_cache(maxsize=None)
def _get_dispatch():
    mesh = plsc.VectorSubcoreMesh(core_axis_name="c", subcore_axis_name="s")
    return pl.kernel(
        _scatter_body,
        mesh=mesh,
        out_type=[
            jax.ShapeDtypeStruct((T, D), jnp.float32),
            jax.ShapeDtypeStruct((T, 128), jnp.float32),
        ],
        scratch_types=[
            pltpu.VMEM((TW,), jnp.int32),
            pltpu.VMEM((TW, D), jnp.float32),
            pltpu.VMEM((TW, 128), jnp.float32),
            pltpu.SemaphoreType.DMA,
        ],
    )


def _gmm_body(tg_ref, tp_ref, rlo_ref, rhi_ref,
              xs_ref, wsort_ref, we_ref, gam_ref, t2_ref, out_ref):
    i = pl.program_id(0)
    phys = tp_ref[i]
    lo = rlo_ref[i]
    hi = rhi_ref[i]
    rows = phys * BM + lax.broadcasted_iota(jnp.int32, (BM, 1), 0)
    mask = (rows >= lo) & (rows < hi)
    xm = jnp.where(mask, xs_ref[...], 0.0).astype(jnp.bfloat16)
    y = lax.dot_general(xm, we_ref[0].astype(jnp.bfloat16),
                        (((1,), (1,)), ((), ())),
                        preferred_element_type=jnp.float32)
    gamma = gam_ref[0, 0, 0]
    w_col = wsort_ref[:, 0:1]
    bias = jnp.where(mask, t2_ref[0, 0][None, :], 0.0)
    contrib = w_col * (gamma * y + bias)
    first = jnp.logical_or(i == 0, phys != tp_ref[jnp.maximum(i - 1, 0)])

    @pl.when(first)
    def _set():
        out_ref[...] = contrib

    @pl.when(jnp.logical_not(first))
    def _acc():
        out_ref[...] += contrib


def _grouped_matmul(tile_g, tile_p, row_lo, row_hi, xs_sorted, wsort,
                    We, gam, table2_3d):
    grid_spec = pltpu.PrefetchScalarGridSpec(
        num_scalar_prefetch=4,
        grid=(NLOG,),
        in_specs=[
            pl.BlockSpec((BM, D), lambda i, tg, tp, rlo, rhi: (tp[i], 0)),
            pl.BlockSpec((BM, 128), lambda i, tg, tp, rlo, rhi: (tp[i], 0)),
            pl.BlockSpec((1, D, D), lambda i, tg, tp, rlo, rhi: (tg[i], 0, 0)),
            pl.BlockSpec((1, 1, 128), lambda i, tg, tp, rlo, rhi: (tg[i], 0, 0)),
            pl.BlockSpec((1, 1, D), lambda i, tg, tp, rlo, rhi: (tg[i], 0, 0)),
        ],
        out_specs=pl.BlockSpec((BM, D), lambda i, tg, tp, rlo, rhi: (tp[i], 0)),
    )
    return pl.pallas_call(
        _gmm_body,
        grid_spec=grid_spec,
        out_shape=jax.ShapeDtypeStruct((T, D), jnp.float32),
    )(tile_g, tile_p, row_lo, row_hi, xs_sorted, wsort, We, gam,
      table2_3d)


def _gather_body(ysort_hbm, pos_hbm, out_hbm, pos_v, rows_v, sem):
    wid = lax.axis_index("s") * NC + lax.axis_index("c")
    base = wid * TW
    pltpu.sync_copy(pos_hbm.at[pl.ds(base, TW)], pos_v)
    pltpu.async_copy(ysort_hbm.at[pos_v], rows_v, sem).wait()
    pltpu.sync_copy(rows_v, out_hbm.at[pl.ds(base, TW)])


@functools.lru_cache(maxsize=None)
def _get_undispatch():
    mesh = plsc.VectorSubcoreMesh(core_axis_name="c", subcore_axis_name="s")
    return pl.kernel(
        _gather_body,
        mesh=mesh,
        out_type=jax.ShapeDtypeStruct((T, D), jnp.float32),
        scratch_types=[
            pltpu.VMEM((TW,), jnp.int32),
            pltpu.VMEM((TW, D), jnp.float32),
            pltpu.SemaphoreType.DMA,
        ],
    )


def kernel(x, Ins_tk, Wg, We, be, Wgam, Wbeta, Wr):
    B, C, L = x.shape
    xf = x.reshape(T, D)

    table2_3d, gam = _compute_tables(Ins_tk, Wbeta, Wgam, be)
    router_logits = xf @ Wg.T
    router_gamma = jnp.mean(Ins_tk @ Wr.T, axis=1)[0]
    logits = router_gamma + router_logits

    wrep, eidx3, rank3, hist = _router(logits)

    sz = hist[0, :E].astype(jnp.int32)
    off = jnp.concatenate([jnp.zeros((1,), jnp.int32),
                           jnp.cumsum(sz)[:-1].astype(jnp.int32)])
    end = off + sz
    t_lo = off // BM
    t_hi = jnp.where(sz > 0, (end - 1) // BM, t_lo - 1)
    n = t_hi - t_lo + 1
    starts = jnp.concatenate([jnp.zeros((1,), jnp.int32),
                              jnp.cumsum(n)[:-1].astype(jnp.int32)])
    total = jnp.sum(n)
    i = jnp.arange(NLOG, dtype=jnp.int32)
    g_i = jnp.sum((starts[None, :] <= i[:, None]).astype(jnp.int32),
                  axis=1) - 1
    one_g = (jnp.arange(E, dtype=jnp.int32)[None, :] == g_i[:, None])

    def _pick(v):
        return jnp.sum(jnp.where(one_g, v[None, :], 0), axis=1)

    phys_i = _pick(t_lo) + (i - _pick(starts))
    off_i, end_i = _pick(off), _pick(end)
    valid = i < total
    tile_p = jnp.where(valid, phys_i, P - 1).astype(jnp.int32)
    tile_g = jnp.where(valid, g_i, 0).astype(jnp.int32)
    row_lo = jnp.where(valid, jnp.maximum(off_i, phys_i * BM), 0)
    row_hi = jnp.where(valid, jnp.minimum(end_i, (phys_i + 1) * BM), 0)

    offp = jnp.pad(off.astype(jnp.float32)[None, :], ((0, 0), (0, 128 - E)))
    pos3 = _compute_pos(eidx3, rank3, offp)
    pos = pos3.reshape(T)

    xs_sorted, wsort = _get_dispatch()(xf, wrep, pos)
    ysorted = _grouped_matmul(tile_g, tile_p,
                              row_lo.astype(jnp.int32),
                              row_hi.astype(jnp.int32),
                              xs_sorted, wsort, We,
                              gam, table2_3d)
    out = _get_undispatch()(ysorted, pos)
    return out.reshape(B, C, D)


# speedup vs baseline: 1.3134x; 1.0430x over previous
import functools

import jax
import jax.numpy as jnp
from jax import lax
from jax.experimental import pallas as pl
from jax.experimental.pallas import tpu as pltpu
from jax.experimental.pallas import tpu_sc as plsc

E = 8
D = 768
T = 2048
BT = 256
BM = 256
P = T // BM
NLOG = P + E - 1

NC, NS = 2, 16
NW = NC * NS
TW = T // NW


def _tables_body(ins_ref, wbeta_ref, wgam_ref, be_ref, table2_ref, gam_ref):
    ins = ins_ref[0]
    m = jnp.mean(ins, axis=0, keepdims=True)
    beta = jnp.sum(wbeta_ref[0] * m, axis=1)
    gamma = jnp.sum(wgam_ref[0, 0] * m[0])
    table2_ref[0, 0, :] = gamma * be_ref[0, 0] + beta
    gam_ref[...] = jnp.full((1, 1, 128), gamma, dtype=jnp.float32)


def _compute_tables(Ins_tk, Wbeta, Wgam, be):
    return pl.pallas_call(
        _tables_body,
        grid=(E,),
        in_specs=[
            pl.BlockSpec((1, 32, D), lambda e: (0, 0, 0)),
            pl.BlockSpec((1, D, D), lambda e: (e, 0, 0)),
            pl.BlockSpec((1, 1, D), lambda e: (e, 0, 0)),
            pl.BlockSpec((1, 1, D), lambda e: (e, 0, 0)),
        ],
        out_specs=[
            pl.BlockSpec((1, 1, D), lambda e: (e, 0, 0)),
            pl.BlockSpec((1, 1, 128), lambda e: (e, 0, 0)),
        ],
        out_shape=[
            jax.ShapeDtypeStruct((E, 1, D), jnp.float32),
            jax.ShapeDtypeStruct((E, 1, 128), jnp.float32),
        ],
    )(Ins_tk, Wbeta, Wgam.reshape(E, 1, D), be.reshape(E, 1, D))


def _router_body(logits_ref, wrep_ref, pos_ref, hist_ref):
    logits = logits_ref[...]
    mx = jnp.max(logits, axis=1, keepdims=True)
    w = 1.0 / jnp.sum(jnp.exp(logits - mx), axis=1, keepdims=True)
    eidx = jnp.argmax(logits, axis=1)
    onehot = (lax.broadcasted_iota(jnp.int32, (T, E), 1)
              == eidx[:, None]).astype(jnp.bfloat16)

    wrep_ref[...] = jnp.broadcast_to(w, (T, 128))

    ii = lax.broadcasted_iota(jnp.int32, (T, T), 0)
    jj = lax.broadcasted_iota(jnp.int32, (T, T), 1)
    tri = (ii > jj).astype(jnp.bfloat16)
    rank_local = lax.dot_general(
        tri, onehot, (((1,), (0,)), ((), ())),
        preferred_element_type=jnp.float32)
    onef = onehot.astype(jnp.float32)
    hist = jnp.sum(onef, axis=0)
    ee = lax.broadcasted_iota(jnp.int32, (E, E), 0)
    ff = lax.broadcasted_iota(jnp.int32, (E, E), 1)
    off_row = jnp.sum(jnp.where(ee < ff, hist[:, None], 0.0), axis=0)
    pos = jnp.sum((rank_local + off_row[None, :]) * onef, axis=1)
    pos_ref[0, 0, :] = pos.astype(jnp.int32)
    hist_ref[...] = jnp.broadcast_to(jnp.pad(hist, (0, 120)), (1, 128))


def _router(logits):
    return pl.pallas_call(
        _router_body,
        out_shape=[
            jax.ShapeDtypeStruct((T, 128), jnp.float32),
            jax.ShapeDtypeStruct((1, 1, T), jnp.int32),
            jax.ShapeDtypeStruct((1, 128), jnp.float32),
        ],
    )(logits)


def _scatter_body(xs_hbm, wrep_hbm, pos_hbm, xsort_hbm, wsort_hbm,
                  pos_v, rows_v, wrow_v, sem):
    wid = lax.axis_index("s") * NC + lax.axis_index("c")
    base = wid * TW
    pltpu.sync_copy(pos_hbm.at[pl.ds(base, TW)], pos_v)
    pltpu.sync_copy(xs_hbm.at[pl.ds(base, TW)], rows_v)
    pltpu.sync_copy(wrep_hbm.at[pl.ds(base, TW)], wrow_v)
    cp1 = pltpu.async_copy(rows_v, xsort_hbm.at[pos_v], sem)
    cp2 = pltpu.async_copy(wrow_v, wsort_hbm.at[pos_v], sem)
    cp1.wait()
    cp2.wait()


@functools.lru_cache(maxsize=None)
def _get_dispatch():
    mesh = plsc.VectorSubcoreMesh(core_axis_name="c", subcore_axis_name="s")
    return pl.kernel(
        _scatter_body,
        mesh=mesh,
        out_type=[
            jax.ShapeDtypeStruct((T, D), jnp.float32),
            jax.ShapeDtypeStruct((T, 128), jnp.float32),
        ],
        scratch_types=[
            pltpu.VMEM((TW,), jnp.int32),
            pltpu.VMEM((TW, D), jnp.float32),
            pltpu.VMEM((TW, 128), jnp.float32),
            pltpu.SemaphoreType.DMA,
        ],
    )


def _gmm_body(tg_ref, tp_ref, rlo_ref, rhi_ref,
              xs_ref, wsort_ref, we_ref, gam_ref, t2_ref, out_ref):
    i = pl.program_id(0)
    phys = tp_ref[i]
    lo = rlo_ref[i]
    hi = rhi_ref[i]
    rows = phys * BM + lax.broadcasted_iota(jnp.int32, (BM, 1), 0)
    mask = (rows >= lo) & (rows < hi)
    xm = jnp.where(mask, xs_ref[...], 0.0).astype(jnp.bfloat16)
    y = lax.dot_general(xm, we_ref[0].astype(jnp.bfloat16),
                        (((1,), (1,)), ((), ())),
                        preferred_element_type=jnp.float32)
    gamma = gam_ref[0, 0, 0]
    w_col = wsort_ref[:, 0:1]
    bias = jnp.where(mask, t2_ref[0, 0][None, :], 0.0)
    contrib = w_col * (gamma * y + bias)
    first = jnp.logical_or(i == 0, phys != tp_ref[jnp.maximum(i - 1, 0)])

    @pl.when(first)
    def _set():
        out_ref[...] = contrib

    @pl.when(jnp.logical_not(first))
    def _acc():
        out_ref[...] += contrib


def _grouped_matmul(tile_g, tile_p, row_lo, row_hi, xs_sorted, wsort,
                    We, gam, table2_3d):
    grid_spec = pltpu.PrefetchScalarGridSpec(
        num_scalar_prefetch=4,
        grid=(NLOG,),
        in_specs=[
            pl.BlockSpec((BM, D), lambda i, tg, tp, rlo, rhi: (tp[i], 0)),
            pl.BlockSpec((BM, 128), lambda i, tg, tp, rlo, rhi: (tp[i], 0)),
            pl.BlockSpec((1, D, D), lambda i, tg, tp, rlo, rhi: (tg[i], 0, 0)),
            pl.BlockSpec((1, 1, 128), lambda i, tg, tp, rlo, rhi: (tg[i], 0, 0)),
            pl.BlockSpec((1, 1, D), lambda i, tg, tp, rlo, rhi: (tg[i], 0, 0)),
        ],
        out_specs=pl.BlockSpec((BM, D), lambda i, tg, tp, rlo, rhi: (tp[i], 0)),
    )
    return pl.pallas_call(
        _gmm_body,
        grid_spec=grid_spec,
        out_shape=jax.ShapeDtypeStruct((T, D), jnp.float32),
    )(tile_g, tile_p, row_lo, row_hi, xs_sorted, wsort, We, gam,
      table2_3d)


def _gather_body(ysort_hbm, pos_hbm, out_hbm, pos_v, rows_v, sem):
    wid = lax.axis_index("s") * NC + lax.axis_index("c")
    base = wid * TW
    pltpu.sync_copy(pos_hbm.at[pl.ds(base, TW)], pos_v)
    pltpu.async_copy(ysort_hbm.at[pos_v], rows_v, sem).wait()
    pltpu.sync_copy(rows_v, out_hbm.at[pl.ds(base, TW)])


@functools.lru_cache(maxsize=None)
def _get_undispatch():
    mesh = plsc.VectorSubcoreMesh(core_axis_name="c", subcore_axis_name="s")
    return pl.kernel(
        _gather_body,
        mesh=mesh,
        out_type=jax.ShapeDtypeStruct((T, D), jnp.float32),
        scratch_types=[
            pltpu.VMEM((TW,), jnp.int32),
            pltpu.VMEM((TW, D), jnp.float32),
            pltpu.SemaphoreType.DMA,
        ],
    )


def kernel(x, Ins_tk, Wg, We, be, Wgam, Wbeta, Wr):
    B, C, L = x.shape
    xf = x.reshape(T, D)

    table2_3d, gam = _compute_tables(Ins_tk, Wbeta, Wgam, be)
    router_logits = xf @ Wg.T
    router_gamma = jnp.mean(Ins_tk @ Wr.T, axis=1)[0]
    logits = router_gamma + router_logits

    wrep, pos3, hist = _router(logits)

    sz = hist[0, :E].astype(jnp.int32)
    off = jnp.concatenate([jnp.zeros((1,), jnp.int32),
                           jnp.cumsum(sz)[:-1].astype(jnp.int32)])
    end = off + sz
    t_lo = off // BM
    t_hi = jnp.where(sz > 0, (end - 1) // BM, t_lo - 1)
    n = t_hi - t_lo + 1
    starts = jnp.concatenate([jnp.zeros((1,), jnp.int32),
                              jnp.cumsum(n)[:-1].astype(jnp.int32)])
    total = jnp.sum(n)
    i = jnp.arange(NLOG, dtype=jnp.int32)
    g_i = jnp.sum((starts[None, :] <= i[:, None]).astype(jnp.int32),
                  axis=1) - 1
    one_g = (jnp.arange(E, dtype=jnp.int32)[None, :] == g_i[:, None])

    def _pick(v):
        return jnp.sum(jnp.where(one_g, v[None, :], 0), axis=1)

    phys_i = _pick(t_lo) + (i - _pick(starts))
    off_i, end_i = _pick(off), _pick(end)
    valid = i < total
    tile_p = jnp.where(valid, phys_i, P - 1).astype(jnp.int32)
    tile_g = jnp.where(valid, g_i, 0).astype(jnp.int32)
    row_lo = jnp.where(valid, jnp.maximum(off_i, phys_i * BM), 0)
    row_hi = jnp.where(valid, jnp.minimum(end_i, (phys_i + 1) * BM), 0)

    pos = pos3.reshape(T)

    xs_sorted, wsort = _get_dispatch()(xf, wrep, pos)
    ysorted = _grouped_matmul(tile_g, tile_p,
                              row_lo.astype(jnp.int32),
                              row_hi.astype(jnp.int32),
                              xs_sorted, wsort, We,
                              gam, table2_3d)
    out = _get_undispatch()(ysorted, pos)
    return out.reshape(B, C, D)


# speedup vs baseline: 1.4231x; 1.0835x over previous
import functools

import jax
import jax.numpy as jnp
from jax import lax
from jax.experimental import pallas as pl
from jax.experimental.pallas import tpu as pltpu
from jax.experimental.pallas import tpu_sc as plsc

E = 8
D = 768
T = 2048
BT = 256
BM = 256
P = T // BM
NLOG = P + E - 1

NC, NS = 2, 16
NW = NC * NS
TW = T // NW


def _tables_body(ins_ref, wbeta_ref, wgam_ref, be_ref, table2_ref, gam_ref):
    ins = ins_ref[0]
    m = jnp.mean(ins, axis=0, keepdims=True)
    beta = jnp.sum(wbeta_ref[0] * m, axis=1)
    gamma = jnp.sum(wgam_ref[0, 0] * m[0])
    table2_ref[0, 0, :] = gamma * be_ref[0, 0] + beta
    gam_ref[...] = jnp.full((1, 1, 128), gamma, dtype=jnp.float32)


def _compute_tables(Ins_tk, Wbeta, Wgam, be):
    return pl.pallas_call(
        _tables_body,
        grid=(E,),
        in_specs=[
            pl.BlockSpec((1, 32, D), lambda e: (0, 0, 0)),
            pl.BlockSpec((1, D, D), lambda e: (e, 0, 0)),
            pl.BlockSpec((1, 1, D), lambda e: (e, 0, 0)),
            pl.BlockSpec((1, 1, D), lambda e: (e, 0, 0)),
        ],
        out_specs=[
            pl.BlockSpec((1, 1, D), lambda e: (e, 0, 0)),
            pl.BlockSpec((1, 1, 128), lambda e: (e, 0, 0)),
        ],
        out_shape=[
            jax.ShapeDtypeStruct((E, 1, D), jnp.float32),
            jax.ShapeDtypeStruct((E, 1, 128), jnp.float32),
        ],
    )(Ins_tk, Wbeta, Wgam.reshape(E, 1, D), be.reshape(E, 1, D))


def _router_body(logits_ref, wrep_ref, pos_ref, tbl_ref):
    logits = logits_ref[...]
    mx = jnp.max(logits, axis=1, keepdims=True)
    w = 1.0 / jnp.sum(jnp.exp(logits - mx), axis=1, keepdims=True)
    eidx = jnp.argmax(logits, axis=1)
    wrep_ref[...] = jnp.broadcast_to(w, (T, 128))

    ii = lax.broadcasted_iota(jnp.int32, (BT, BT), 0)
    jj = lax.broadcasted_iota(jnp.int32, (BT, BT), 1)
    tri = (ii > jj).astype(jnp.bfloat16)
    ecols = lax.broadcasted_iota(jnp.int32, (BT, E), 1)
    hist = jnp.zeros((E,), jnp.float32)
    ranks, ohs = [], []
    for b in range(T // BT):
        oh = (ecols == eidx[b * BT:(b + 1) * BT, None]).astype(jnp.bfloat16)
        ohf = oh.astype(jnp.float32)
        r = lax.dot_general(tri, oh, (((1,), (0,)), ((), ())),
                            preferred_element_type=jnp.float32)
        ranks.append(r + hist[None, :])
        ohs.append(ohf)
        hist = hist + jnp.sum(ohf, axis=0)
    ee = lax.broadcasted_iota(jnp.int32, (E, E), 0)
    ff = lax.broadcasted_iota(jnp.int32, (E, E), 1)
    off = jnp.sum(jnp.where(ee < ff, hist[:, None], 0.0), axis=0)
    for b in range(T // BT):
        pos_b = jnp.sum((ranks[b] + off[None, :]) * ohs[b], axis=1)
        pos_ref[0, 0, b * BT:(b + 1) * BT] = pos_b.astype(jnp.int32)

    szi = hist.astype(jnp.int32)
    offi = off.astype(jnp.int32)
    endi = offi + szi
    t_lo = offi // BM
    t_hi = jnp.where(szi > 0, (endi - 1) // BM, t_lo - 1)
    n = t_hi - t_lo + 1
    starts = jnp.sum(jnp.where(ee < ff, n[:, None], 0), axis=0)
    total = jnp.sum(n)
    im = lax.broadcasted_iota(jnp.int32, (NLOG, E), 0)
    g = jnp.sum((starts[None, :] <= im).astype(jnp.int32), axis=1) - 1
    one_g = (lax.broadcasted_iota(jnp.int32, (NLOG, E), 1) == g[:, None])

    def _pick(v):
        return jnp.sum(jnp.where(one_g, v[None, :], 0), axis=1)

    i_1d = im[:, 0]
    phys = _pick(t_lo) + (i_1d - _pick(starts))
    valid = i_1d < total
    tp = jnp.where(valid, phys, P - 1)
    tg = jnp.where(valid, g, 0)
    rlo = jnp.where(valid, jnp.maximum(_pick(offi), phys * BM), 0)
    rhi = jnp.where(valid, jnp.minimum(_pick(endi), (phys + 1) * BM), 0)
    rows = [jnp.pad(v, (0, 128 - NLOG))[None, :] for v in (tg, tp, rlo, rhi)]
    tbl_ref[...] = jnp.concatenate(rows, axis=0)


def _router(logits):
    return pl.pallas_call(
        _router_body,
        out_shape=[
            jax.ShapeDtypeStruct((T, 128), jnp.float32),
            jax.ShapeDtypeStruct((1, 1, T), jnp.int32),
            jax.ShapeDtypeStruct((4, 128), jnp.int32),
        ],
    )(logits)


def _scatter_body(xs_hbm, wrep_hbm, pos_hbm, xsort_hbm, wsort_hbm,
                  pos_v, rows_v, wrow_v, sem):
    wid = lax.axis_index("s") * NC + lax.axis_index("c")
    base = wid * TW
    pltpu.sync_copy(pos_hbm.at[pl.ds(base, TW)], pos_v)
    pltpu.sync_copy(xs_hbm.at[pl.ds(base, TW)], rows_v)
    pltpu.sync_copy(wrep_hbm.at[pl.ds(base, TW)], wrow_v)
    cp1 = pltpu.async_copy(rows_v, xsort_hbm.at[pos_v], sem)
    cp2 = pltpu.async_copy(wrow_v, wsort_hbm.at[pos_v], sem)
    cp1.wait()
    cp2.wait()


@functools.lru_cache(maxsize=None)
def _get_dispatch():
    mesh = plsc.VectorSubcoreMesh(core_axis_name="c", subcore_axis_name="s")
    return pl.kernel(
        _scatter_body,
        mesh=mesh,
        out_type=[
            jax.ShapeDtypeStruct((T, D), jnp.float32),
            jax.ShapeDtypeStruct((T, 128), jnp.float32),
        ],
        scratch_types=[
            pltpu.VMEM((TW,), jnp.int32),
            pltpu.VMEM((TW, D), jnp.float32),
            pltpu.VMEM((TW, 128), jnp.float32),
            pltpu.SemaphoreType.DMA,
        ],
    )


def _gmm_body(tg_ref, tp_ref, rlo_ref, rhi_ref,
              xs_ref, wsort_ref, we_ref, gam_ref, t2_ref, out_ref):
    i = pl.program_id(0)
    phys = tp_ref[i]
    lo = rlo_ref[i]
    hi = rhi_ref[i]
    rows = phys * BM + lax.broadcasted_iota(jnp.int32, (BM, 1), 0)
    mask = (rows >= lo) & (rows < hi)
    xm = jnp.where(mask, xs_ref[...], 0.0).astype(jnp.bfloat16)
    y = lax.dot_general(xm, we_ref[0].astype(jnp.bfloat16),
                        (((1,), (1,)), ((), ())),
                        preferred_element_type=jnp.float32)
    gamma = gam_ref[0, 0, 0]
    w_col = wsort_ref[:, 0:1]
    bias = jnp.where(mask, t2_ref[0, 0][None, :], 0.0)
    contrib = w_col * (gamma * y + bias)
    first = jnp.logical_or(i == 0, phys != tp_ref[jnp.maximum(i - 1, 0)])

    @pl.when(first)
    def _set():
        out_ref[...] = contrib

    @pl.when(jnp.logical_not(first))
    def _acc():
        out_ref[...] += contrib


def _grouped_matmul(tile_g, tile_p, row_lo, row_hi, xs_sorted, wsort,
                    We, gam, table2_3d):
    grid_spec = pltpu.PrefetchScalarGridSpec(
        num_scalar_prefetch=4,
        grid=(NLOG,),
        in_specs=[
            pl.BlockSpec((BM, D), lambda i, tg, tp, rlo, rhi: (tp[i], 0)),
            pl.BlockSpec((BM, 128), lambda i, tg, tp, rlo, rhi: (tp[i], 0)),
            pl.BlockSpec((1, D, D), lambda i, tg, tp, rlo, rhi: (tg[i], 0, 0)),
            pl.BlockSpec((1, 1, 128), lambda i, tg, tp, rlo, rhi: (tg[i], 0, 0)),
            pl.BlockSpec((1, 1, D), lambda i, tg, tp, rlo, rhi: (tg[i], 0, 0)),
        ],
        out_specs=pl.BlockSpec((BM, D), lambda i, tg, tp, rlo, rhi: (tp[i], 0)),
    )
    return pl.pallas_call(
        _gmm_body,
        grid_spec=grid_spec,
        out_shape=jax.ShapeDtypeStruct((T, D), jnp.float32),
    )(tile_g, tile_p, row_lo, row_hi, xs_sorted, wsort, We, gam,
      table2_3d)


def _gather_body(ysort_hbm, pos_hbm, out_hbm, pos_v, rows_v, sem):
    wid = lax.axis_index("s") * NC + lax.axis_index("c")
    base = wid * TW
    pltpu.sync_copy(pos_hbm.at[pl.ds(base, TW)], pos_v)
    pltpu.async_copy(ysort_hbm.at[pos_v], rows_v, sem).wait()
    pltpu.sync_copy(rows_v, out_hbm.at[pl.ds(base, TW)])


@functools.lru_cache(maxsize=None)
def _get_undispatch():
    mesh = plsc.VectorSubcoreMesh(core_axis_name="c", subcore_axis_name="s")
    return pl.kernel(
        _gather_body,
        mesh=mesh,
        out_type=jax.ShapeDtypeStruct((T, D), jnp.float32),
        scratch_types=[
            pltpu.VMEM((TW,), jnp.int32),
            pltpu.VMEM((TW, D), jnp.float32),
            pltpu.SemaphoreType.DMA,
        ],
    )


def kernel(x, Ins_tk, Wg, We, be, Wgam, Wbeta, Wr):
    B, C, L = x.shape
    xf = x.reshape(T, D)

    table2_3d, gam = _compute_tables(Ins_tk, Wbeta, Wgam, be)
    router_logits = xf @ Wg.T
    router_gamma = jnp.mean(Ins_tk @ Wr.T, axis=1)[0]
    logits = router_gamma + router_logits

    wrep, pos3, tbl = _router(logits)
    tile_g = tbl[0, :NLOG]
    tile_p = tbl[1, :NLOG]
    row_lo = tbl[2, :NLOG]
    row_hi = tbl[3, :NLOG]
    pos = pos3.reshape(T)

    xs_sorted, wsort = _get_dispatch()(xf, wrep, pos)
    ysorted = _grouped_matmul(tile_g, tile_p,
                              row_lo.astype(jnp.int32),
                              row_hi.astype(jnp.int32),
                              xs_sorted, wsort, We,
                              gam, table2_3d)
    out = _get_undispatch()(ysorted, pos)
    return out.reshape(B, C, D)
